# baseline (speedup 1.0000x reference)
import jax
import jax.numpy as jnp
from jax.experimental import pallas as pl


def _tc_body(x_ref, w_ref, o_ref):
    inv = 1.0 / (x_ref.shape[1] * pl.num_programs(1))
    j = pl.program_id(1)
    s = jnp.sum(x_ref[...], axis=1)
    part = jax.lax.dot_general(
        s, w_ref[...],
        dimension_numbers=(((1,), (1,)), ((), ())),
        preferred_element_type=jnp.float32,
    ) * inv

    @pl.when(j == 0)
    def _init():
        o_ref[...] = part

    @pl.when(j > 0)
    def _acc():
        o_ref[...] = o_ref[...] + part


def kernel(x, W):
    B, C, H, Wsp = x.shape
    S = H * Wsp
    E = W.shape[0]
    xv = jnp.transpose(x, (0, 2, 3, 1)).reshape(B, S, C)
    Bblk = 8
    NS = 4
    return pl.pallas_call(
        _tc_body,
        grid=(B // Bblk, NS),
        in_specs=[
            pl.BlockSpec((Bblk, S // NS, C), lambda i, j: (i, j, 0)),
            pl.BlockSpec((E, C), lambda i, j: (0, 0)),
        ],
        out_specs=pl.BlockSpec((Bblk, E), lambda i, j: (i, 0)),
        out_shape=jax.ShapeDtypeStruct((B, E), jnp.float32),
    )(xv, W)
